# baseline (speedup 1.0000x reference)
import jax
import jax.numpy as jnp
from jax import lax
from jax.experimental import pallas as pl
from jax.experimental.pallas import tpu as pltpu
from jax.experimental.pallas import tpu_sc as plsc

NC, NS, L = 2, 16, 16
NW = NC * NS
CHUNK = 32
NSLOT = 6
EPS = 1e-5


def _lane_sum(s):
    idx = lax.iota(jnp.int32, L)
    dnums = lax.GatherDimensionNumbers(offset_dims=(), collapsed_slice_dims=(0,),
                                       start_index_map=(0,))
    for off in (8, 4, 2, 1):
        perm = lax.bitwise_xor(idx, off)
        s = s + lax.gather(s, perm[:, None], dnums, slice_sizes=(1,),
                           mode=lax.GatherScatterMode.PROMISE_IN_BOUNDS)
    return s


def _rsqrt_vec(v):
    i = lax.bitcast_convert_type(v, jnp.int32)
    y = lax.bitcast_convert_type(jnp.int32(0x5F3759DF) - lax.shift_right_arithmetic(i, 1),
                                 jnp.float32)
    half = v * 0.5
    for _ in range(3):
        y = y * (1.5 - half * y * y)
    return y


def _sc_scatter_call(me, e, p, midx, uidx, gamma, beta, B, T, K, NM, NU):
    m_per_w = (B * NM) // NW
    u_per_w = (B * NU) // NW
    n_mchunks = m_per_w // CHUNK
    n_uchunks = u_per_w // CHUNK
    n_mgroups = n_mchunks // NSLOT
    n_ugroups = (n_uchunks - 1) // 3
    inv_k = jnp.float32(1.0 / K)

    def body(me_hbm, e_hbm, p_hbm, midx_hbm, uidx_hbm, g_hbm, b_hbm, out_hbm,
             D, idxm, idxu, gv, bv, *sems):
        sem_g = sems[:NSLOT]
        sem_s = sems[NSLOT:2 * NSLOT]
        sem_ug = sems[2 * NSLOT:2 * NSLOT + 3]
        sem_us = sems[2 * NSLOT + 3:]
        wid = lax.axis_index("s") * NC + lax.axis_index("c")
        pltpu.sync_copy(g_hbm, gv)
        pltpu.sync_copy(b_hbm, bv)

        def adjust_idx(ref, slot, base, rows_per_batch):
            bofs = (base // rows_per_batch) * T
            for i in range(CHUNK // L):
                sl = pl.ds(i * L, L)
                ref[slot, sl] = ref[slot, sl] + bofs

        def m_base(c):
            return wid * m_per_w + c * CHUNK

        def m_gather(c, slot):
            base = m_base(c)
            return (pltpu.make_async_copy(midx_hbm.at[pl.ds(base, CHUNK)],
                                          idxm.at[slot], sem_g[slot]),
                    pltpu.make_async_copy(me_hbm.at[pl.ds(base, CHUNK)],
                                          D.at[slot], sem_g[slot]))

        def m_scatter(slot):
            return pltpu.make_async_copy(D.at[slot], out_hbm.at[idxm.at[slot]],
                                         sem_s[slot])

        for b in range(NSLOT):
            for d in m_gather(b, b):
                d.start()

        def mgroup(g, carry):
            for b in range(NSLOT):
                c = g * NSLOT + b
                for d in m_gather(c, b):
                    d.wait()
                adjust_idx(idxm, b, m_base(c), NM)
                m_scatter(b).start()
                sl = (b + 3) % NSLOT
                cprev = c - 3

                @pl.when(cprev >= 0)
                def _():
                    m_scatter(sl).wait()

                @pl.when(jnp.logical_and(cprev >= 0, c + 3 < n_mchunks))
                def _():
                    for d in m_gather(c + 3, sl):
                        d.start()

            return carry

        lax.fori_loop(0, n_mgroups, mgroup, 0)
        for b in range(3):
            m_scatter((n_mchunks - 3 + b) % NSLOT).wait()

        def u_base(c):
            return wid * u_per_w + c * CHUNK

        def u_gather(c, pr):
            base = u_base(c)
            return (pltpu.make_async_copy(uidx_hbm.at[pl.ds(base, CHUNK)],
                                          idxu.at[pr], sem_ug[pr]),
                    pltpu.make_async_copy(e_hbm.at[pl.ds(base, CHUNK)],
                                          D.at[2 * pr], sem_ug[pr]),
                    pltpu.make_async_copy(p_hbm.at[pl.ds(base, CHUNK)],
                                          D.at[2 * pr + 1], sem_ug[pr]))

        def u_scatter(pr):
            return pltpu.make_async_copy(D.at[2 * pr], out_hbm.at[idxu.at[pr]],
                                         sem_us[pr])

        NROW = 4

        def u_compute(c, pr):
            adjust_idx(idxu, pr, u_base(c), NU)

            def rowgrp(rr, rcarry):
                rows = tuple(NROW * rr + j for j in range(NROW))
                s1 = [jnp.zeros((L,), jnp.float32) for _ in rows]
                s2 = [jnp.zeros((L,), jnp.float32) for _ in rows]
                for i in range(K // L):
                    sl = pl.ds(i * L, L)
                    for j, r in enumerate(rows):
                        xv = D[2 * pr, r, sl] + D[2 * pr + 1, r, sl]
                        D[2 * pr, r, sl] = xv
                        s1[j] = s1[j] + xv
                        s2[j] = s2[j] + xv * xv
                mvec = [None] * NROW
                rstd = [None] * NROW
                for j in range(NROW):
                    mvec[j] = _lane_sum(s1[j]) * inv_k
                    var = _lane_sum(s2[j]) * inv_k - mvec[j] * mvec[j]
                    rstd[j] = _rsqrt_vec(var + EPS)
                for i in range(K // L):
                    sl = pl.ds(i * L, L)
                    gsl = gv[sl]
                    bsl = bv[sl]
                    for j, r in enumerate(rows):
                        D[2 * pr, r, sl] = ((D[2 * pr, r, sl] - mvec[j])
                                            * rstd[j] * gsl + bsl)
                return rcarry

            lax.fori_loop(0, CHUNK // NROW, rowgrp, 0)

        def u_step(c, pr):
            for d in u_gather(c, pr):
                d.wait()
            npr = (pr + 1) % 3
            cnext = c + 1

            @pl.when(c - 2 >= 0)
            def _():
                u_scatter(npr).wait()

            @pl.when(jnp.logical_and(cnext >= 2, cnext < n_uchunks))
            def _():
                for d in u_gather(cnext, npr):
                    d.start()

            u_compute(c, pr)
            u_scatter(pr).start()

        for c in range(2):
            for d in u_gather(c, c):
                d.start()

        def ugroup(g, carry):
            for b in range(3):
                u_step(g * 3 + b, b)
            return carry

        lax.fori_loop(0, n_ugroups, ugroup, 0)
        u_step(n_uchunks - 1, (n_uchunks - 1) % 3)
        for j in range(2):
            u_scatter((n_uchunks - 2 + j) % 3).wait()

    mesh = plsc.VectorSubcoreMesh(core_axis_name="c", subcore_axis_name="s")
    f = pl.kernel(
        body,
        out_type=jax.ShapeDtypeStruct((B * T, K), jnp.float32),
        mesh=mesh,
        scratch_types=[
            pltpu.VMEM((NSLOT, CHUNK, K), jnp.float32),
            pltpu.VMEM((NSLOT, CHUNK), jnp.int32),
            pltpu.VMEM((3, CHUNK), jnp.int32),
            pltpu.VMEM((K,), jnp.float32),
            pltpu.VMEM((K,), jnp.float32),
        ] + [pltpu.SemaphoreType.DMA] * (2 * NSLOT + 6),
    )
    return f(me, e, p, midx, uidx, gamma, beta)


def kernel(encoder_output, mask_embedding, unmasked_positions, mask_id, unmask_id,
           gamma, beta):
    B, NU, K = encoder_output.shape
    NM = mask_embedding.shape[1]
    T = NM + NU
    me = mask_embedding.reshape(B * NM, K)
    e = encoder_output.reshape(B * NU, K)
    p = unmasked_positions.reshape(B * NU, K)
    midx = mask_id.reshape(B * NM)
    uidx = unmask_id.reshape(B * NU)
    out = _sc_scatter_call(me, e, p, midx, uidx, gamma, beta, B, T, K, NM, NU)
    return out.reshape(B, T, K)


# speedup vs baseline: 1.3220x; 1.3220x over previous
import jax
import jax.numpy as jnp
from jax import lax
from jax.experimental import pallas as pl
from jax.experimental.pallas import tpu as pltpu
from jax.experimental.pallas import tpu_sc as plsc

NC, NS, L = 2, 16, 16
NW = NC * NS
CHUNK = 32
MSLOT = 3
NROW = 2
EPS = 1e-5


def _lane_sum(s):
    idx = lax.iota(jnp.int32, L)
    dnums = lax.GatherDimensionNumbers(offset_dims=(), collapsed_slice_dims=(0,),
                                       start_index_map=(0,))
    for off in (8, 4, 2, 1):
        perm = lax.bitwise_xor(idx, off)
        s = s + lax.gather(s, perm[:, None], dnums, slice_sizes=(1,),
                           mode=lax.GatherScatterMode.PROMISE_IN_BOUNDS)
    return s


def _rsqrt_vec(v):
    i = lax.bitcast_convert_type(v, jnp.int32)
    y = lax.bitcast_convert_type(jnp.int32(0x5F3759DF) - lax.shift_right_arithmetic(i, 1),
                                 jnp.float32)
    half = v * 0.5
    for _ in range(3):
        y = y * (1.5 - half * y * y)
    return y


def _sc_scatter_call(me, e, p, midx, uidx, gamma, beta, B, T, K, NM, NU):
    m_per_w = (B * NM) // NW
    u_per_w = (B * NU) // NW
    n_mchunks = m_per_w // CHUNK
    n_steps = u_per_w // CHUNK
    inv_k = jnp.float32(1.0 / K)

    def body(me_hbm, e_hbm, p_hbm, midx_hbm, uidx_hbm, g_hbm, b_hbm, out_hbm,
             DM, DU, idxm, idxu, gv, bv, *sems):
        sem_mg = sems[:MSLOT]
        sem_ms = sems[MSLOT:2 * MSLOT]
        sem_ug = sems[2 * MSLOT:2 * MSLOT + 2]
        sem_us = sems[2 * MSLOT + 2:]
        wid = lax.axis_index("s") * NC + lax.axis_index("c")
        pltpu.sync_copy(g_hbm, gv)
        pltpu.sync_copy(b_hbm, bv)

        def adjust_idx(ref, slot, base, rows_per_batch):
            bofs = (base // rows_per_batch) * T
            for i in range(CHUNK // L):
                sl = pl.ds(i * L, L)
                ref[slot, sl] = ref[slot, sl] + bofs

        def m_base(c):
            return wid * m_per_w + c * CHUNK

        def m_gather(c, slot):
            base = m_base(c)
            return (pltpu.make_async_copy(midx_hbm.at[pl.ds(base, CHUNK)],
                                          idxm.at[slot], sem_mg[slot]),
                    pltpu.make_async_copy(me_hbm.at[pl.ds(base, CHUNK)],
                                          DM.at[slot], sem_mg[slot]))

        def m_scatter(slot):
            return pltpu.make_async_copy(DM.at[slot], out_hbm.at[idxm.at[slot]],
                                         sem_ms[slot])

        def u_base(c):
            return wid * u_per_w + c * CHUNK

        def u_gather(c, pr):
            base = u_base(c)
            return (pltpu.make_async_copy(uidx_hbm.at[pl.ds(base, CHUNK)],
                                          idxu.at[pr], sem_ug[pr]),
                    pltpu.make_async_copy(e_hbm.at[pl.ds(base, CHUNK)],
                                          DU.at[2 * pr], sem_ug[pr]),
                    pltpu.make_async_copy(p_hbm.at[pl.ds(base, CHUNK)],
                                          DU.at[2 * pr + 1], sem_ug[pr]))

        def u_scatter(pr):
            return pltpu.make_async_copy(DU.at[2 * pr], out_hbm.at[idxu.at[pr]],
                                         sem_us[pr])

        def u_compute(c, pr):
            adjust_idx(idxu, pr, u_base(c), NU)

            def rowgrp(rr, rcarry):
                rows = tuple(NROW * rr + j for j in range(NROW))
                s1 = [jnp.zeros((L,), jnp.float32) for _ in rows]
                s2 = [jnp.zeros((L,), jnp.float32) for _ in rows]
                for i in range(K // L):
                    sl = pl.ds(i * L, L)
                    for j, r in enumerate(rows):
                        xv = DU[2 * pr, r, sl] + DU[2 * pr + 1, r, sl]
                        DU[2 * pr, r, sl] = xv
                        s1[j] = s1[j] + xv
                        s2[j] = s2[j] + xv * xv
                mvec = [None] * NROW
                rstd = [None] * NROW
                for j in range(NROW):
                    mvec[j] = _lane_sum(s1[j]) * inv_k
                    var = _lane_sum(s2[j]) * inv_k - mvec[j] * mvec[j]
                    rstd[j] = _rsqrt_vec(var + EPS)
                for i in range(K // L):
                    sl = pl.ds(i * L, L)
                    gsl = gv[sl]
                    bsl = bv[sl]
                    for j, r in enumerate(rows):
                        DU[2 * pr, r, sl] = ((DU[2 * pr, r, sl] - mvec[j])
                                             * rstd[j] * gsl + bsl)
                return rcarry

            lax.fori_loop(0, CHUNK // NROW, rowgrp, 0)

        for b in range(MSLOT):
            for d in m_gather(b, b):
                d.start()
        for c in range(2):
            for d in u_gather(c, c):
                d.start()

        def step(s, pr):
            for b in range(MSLOT):
                c = s * MSLOT + b
                for d in m_gather(c, b):
                    d.wait()
                adjust_idx(idxm, b, m_base(c), NM)
                m_scatter(b).start()

            for d in u_gather(s, pr):
                d.wait()
            npr = 1 - pr

            @pl.when(s >= 1)
            def _():
                u_scatter(npr).wait()

            @pl.when(jnp.logical_and(s + 1 >= 2, s + 1 < n_steps))
            def _():
                for d in u_gather(s + 1, npr):
                    d.start()

            u_compute(s, pr)
            u_scatter(pr).start()

            for b in range(MSLOT):
                c = s * MSLOT + b
                m_scatter(b).wait()

                @pl.when(s + 1 < n_steps)
                def _():
                    for d in m_gather(c + MSLOT, b):
                        d.start()

        def group(g, carry):
            step(2 * g, 0)
            step(2 * g + 1, 1)
            return carry

        lax.fori_loop(0, n_steps // 2, group, 0)
        u_scatter((n_steps - 1) % 2).wait()

    mesh = plsc.VectorSubcoreMesh(core_axis_name="c", subcore_axis_name="s")
    f = pl.kernel(
        body,
        out_type=jax.ShapeDtypeStruct((B * T, K), jnp.float32),
        mesh=mesh,
        scratch_types=[
            pltpu.VMEM((MSLOT, CHUNK, K), jnp.float32),
            pltpu.VMEM((4, CHUNK, K), jnp.float32),
            pltpu.VMEM((MSLOT, CHUNK), jnp.int32),
            pltpu.VMEM((2, CHUNK), jnp.int32),
            pltpu.VMEM((K,), jnp.float32),
            pltpu.VMEM((K,), jnp.float32),
        ] + [pltpu.SemaphoreType.DMA] * (2 * MSLOT + 4),
    )
    return f(me, e, p, midx, uidx, gamma, beta)


def kernel(encoder_output, mask_embedding, unmasked_positions, mask_id, unmask_id,
           gamma, beta):
    B, NU, K = encoder_output.shape
    NM = mask_embedding.shape[1]
    T = NM + NU
    me = mask_embedding.reshape(B * NM, K)
    e = encoder_output.reshape(B * NU, K)
    p = unmasked_positions.reshape(B * NU, K)
    midx = mask_id.reshape(B * NM)
    uidx = unmask_id.reshape(B * NU)
    out = _sc_scatter_call(me, e, p, midx, uidx, gamma, beta, B, T, K, NM, NU)
    return out.reshape(B, T, K)


# speedup vs baseline: 1.4290x; 1.0809x over previous
import jax
import jax.numpy as jnp
from jax import lax
from jax.experimental import pallas as pl
from jax.experimental.pallas import tpu as pltpu
from jax.experimental.pallas import tpu_sc as plsc

NC, NS, L = 2, 16, 16
NW = NC * NS
CHUNK = 32
MSLOT = 3
NROW = 2
EPS = 1e-5


def _lane_sum(s):
    idx = lax.iota(jnp.int32, L)
    dnums = lax.GatherDimensionNumbers(offset_dims=(), collapsed_slice_dims=(0,),
                                       start_index_map=(0,))
    for off in (8, 4, 2, 1):
        perm = lax.bitwise_xor(idx, off)
        s = s + lax.gather(s, perm[:, None], dnums, slice_sizes=(1,),
                           mode=lax.GatherScatterMode.PROMISE_IN_BOUNDS)
    return s


def _rsqrt_vec(v):
    i = lax.bitcast_convert_type(v, jnp.int32)
    y = lax.bitcast_convert_type(jnp.int32(0x5F3759DF) - lax.shift_right_arithmetic(i, 1),
                                 jnp.float32)
    half = v * 0.5
    for _ in range(3):
        y = y * (1.5 - half * y * y)
    return y


def _sc_scatter_call(me, e, p, midx, uidx, gamma, beta, B, T, K, NM, NU):
    m_per_w = (B * NM) // NW
    u_per_w = (B * NU) // NW
    n_mchunks = m_per_w // CHUNK
    n_steps = u_per_w // CHUNK
    inv_k = jnp.float32(1.0 / K)

    def body(me_hbm, e_hbm, p_hbm, midx_hbm, uidx_hbm, g_hbm, b_hbm, out_hbm,
             DM, DU, idxm, idxu, idxu3, gv, bv, *sems):
        sem_mg = sems[:MSLOT]
        sem_ms = sems[MSLOT:2 * MSLOT]
        sem_ug = sems[2 * MSLOT:2 * MSLOT + 2]
        sem_us = sems[2 * MSLOT + 2:]
        wid = lax.axis_index("s") * NC + lax.axis_index("c")
        pltpu.sync_copy(g_hbm, gv)
        pltpu.sync_copy(b_hbm, bv)

        def adjust_idx(ref, slot, base, rows_per_batch):
            bofs = (base // rows_per_batch) * T
            for i in range(CHUNK // L):
                sl = pl.ds(i * L, L)
                ref[slot, sl] = ref[slot, sl] + bofs

        def m_base(c):
            return wid * m_per_w + c * CHUNK

        def m_gather(c, slot):
            base = m_base(c)
            return (pltpu.make_async_copy(midx_hbm.at[pl.ds(base, CHUNK)],
                                          idxm.at[slot], sem_mg[slot]),
                    pltpu.make_async_copy(me_hbm.at[pl.ds(base, CHUNK)],
                                          DM.at[slot], sem_mg[slot]))

        def m_scatter(slot):
            return pltpu.make_async_copy(DM.at[slot], out_hbm.at[idxm.at[slot]],
                                         sem_ms[slot])

        def u_base(c):
            return wid * u_per_w + c * CHUNK

        def u_gather(c, pr):
            base = u_base(c)
            return (pltpu.make_async_copy(uidx_hbm.at[pl.ds(base, CHUNK)],
                                          idxu.at[pr], sem_ug[pr]),
                    pltpu.make_async_copy(e_hbm.at[pl.ds(base, CHUNK)],
                                          DU.at[2 * pr], sem_ug[pr]),
                    pltpu.make_async_copy(p_hbm.at[pl.ds(base, CHUNK)],
                                          DU.at[2 * pr + 1], sem_ug[pr]))

        HALF = CHUNK // 2

        def u_scatter(pr, h):
            return pltpu.make_async_copy(
                DU.at[2 * pr, pl.ds(h * HALF, HALF)],
                out_hbm.at[idxu3.at[pr, h]], sem_us[pr])

        def u_adjust(c, pr):
            bofs = (u_base(c) // NU) * T
            for h in range(2):
                idxu3[pr, h, :] = idxu[pr, pl.ds(h * L, L)] + bofs

        def u_compute(c, pr, h):

            def rowgrp(rr, rcarry):
                rows = tuple(NROW * rr + j for j in range(NROW))
                s1 = [jnp.zeros((L,), jnp.float32) for _ in rows]
                s2 = [jnp.zeros((L,), jnp.float32) for _ in rows]
                for i in range(K // L):
                    sl = pl.ds(i * L, L)
                    for j, r in enumerate(rows):
                        xv = DU[2 * pr, r, sl] + DU[2 * pr + 1, r, sl]
                        DU[2 * pr, r, sl] = xv
                        s1[j] = s1[j] + xv
                        s2[j] = s2[j] + xv * xv
                mvec = [None] * NROW
                rstd = [None] * NROW
                for j in range(NROW):
                    mvec[j] = _lane_sum(s1[j]) * inv_k
                    var = _lane_sum(s2[j]) * inv_k - mvec[j] * mvec[j]
                    rstd[j] = _rsqrt_vec(var + EPS)
                for i in range(K // L):
                    sl = pl.ds(i * L, L)
                    gsl = gv[sl]
                    bsl = bv[sl]
                    for j, r in enumerate(rows):
                        DU[2 * pr, r, sl] = ((DU[2 * pr, r, sl] - mvec[j])
                                             * rstd[j] * gsl + bsl)
                return rcarry

            lax.fori_loop(h * (HALF // NROW), (h + 1) * (HALF // NROW),
                          rowgrp, 0)

        for b in range(MSLOT):
            for d in m_gather(b, b):
                d.start()
        for c in range(2):
            for d in u_gather(c, c):
                d.start()

        def step(s, pr):
            for b in range(MSLOT):
                c = s * MSLOT + b
                for d in m_gather(c, b):
                    d.wait()
                adjust_idx(idxm, b, m_base(c), NM)
                m_scatter(b).start()

            for d in u_gather(s, pr):
                d.wait()
            npr = 1 - pr

            @pl.when(s >= 1)
            def _():
                for h in range(2):
                    u_scatter(npr, h).wait()

            @pl.when(jnp.logical_and(s + 1 >= 2, s + 1 < n_steps))
            def _():
                for d in u_gather(s + 1, npr):
                    d.start()

            u_adjust(s, pr)
            u_compute(s, pr, 0)
            u_scatter(pr, 0).start()

            for b in range(MSLOT):
                c = s * MSLOT + b
                m_scatter(b).wait()

                @pl.when(s + 1 < n_steps)
                def _():
                    for d in m_gather(c + MSLOT, b):
                        d.start()

            u_compute(s, pr, 1)
            u_scatter(pr, 1).start()

        def group(g, carry):
            step(2 * g, 0)
            step(2 * g + 1, 1)
            return carry

        lax.fori_loop(0, n_steps // 2, group, 0)
        for h in range(2):
            u_scatter((n_steps - 1) % 2, h).wait()

    mesh = plsc.VectorSubcoreMesh(core_axis_name="c", subcore_axis_name="s")
    f = pl.kernel(
        body,
        out_type=jax.ShapeDtypeStruct((B * T, K), jnp.float32),
        mesh=mesh,
        scratch_types=[
            pltpu.VMEM((MSLOT, CHUNK, K), jnp.float32),
            pltpu.VMEM((4, CHUNK, K), jnp.float32),
            pltpu.VMEM((MSLOT, CHUNK), jnp.int32),
            pltpu.VMEM((2, CHUNK), jnp.int32),
            pltpu.VMEM((2, 2, L), jnp.int32),
            pltpu.VMEM((K,), jnp.float32),
            pltpu.VMEM((K,), jnp.float32),
        ] + [pltpu.SemaphoreType.DMA] * (2 * MSLOT + 4),
    )
    return f(me, e, p, midx, uidx, gamma, beta)


def kernel(encoder_output, mask_embedding, unmasked_positions, mask_id, unmask_id,
           gamma, beta):
    B, NU, K = encoder_output.shape
    NM = mask_embedding.shape[1]
    T = NM + NU
    me = mask_embedding.reshape(B * NM, K)
    e = encoder_output.reshape(B * NU, K)
    p = unmasked_positions.reshape(B * NU, K)
    midx = mask_id.reshape(B * NM)
    uidx = unmask_id.reshape(B * NU)
    out = _sc_scatter_call(me, e, p, midx, uidx, gamma, beta, B, T, K, NM, NU)
    return out.reshape(B, T, K)
